# Initial kernel scaffold; baseline (speedup 1.0000x reference)
import jax
import jax.numpy as jnp
from jax.experimental import pallas as pl


def kernel(indices, table):
    raise NotImplementedError("write your pallas kernel here")


# speedup vs baseline: 1.5924x; 1.5924x over previous
import functools

import jax
import jax.numpy as jnp
from jax import lax
from jax.experimental import pallas as pl
from jax.experimental.pallas import tpu as pltpu
from jax.experimental.pallas import tpu_sc as plsc

_INFO = plsc.get_sparse_core_info()
_NC = _INFO.num_cores
_NS = _INFO.num_subcores
_NW = _NC * _NS


def _build_gather(n_total: int, d: int, n_chunks: int, rows_per_chunk: int,
                  ring: int):
    n_per_w = n_total // _NW
    assert n_per_w == n_chunks * rows_per_chunk
    assert n_chunks % ring == 0
    n_groups = n_chunks // ring
    mesh = plsc.VectorSubcoreMesh(core_axis_name="c", subcore_axis_name="s")

    @functools.partial(
        pl.kernel,
        mesh=mesh,
        out_type=jax.ShapeDtypeStruct((n_total, d), jnp.float32),
        scratch_types=(
            [pltpu.VMEM((n_chunks, rows_per_chunk), jnp.int32)]
            + [pltpu.VMEM((rows_per_chunk, d), jnp.float32) for _ in range(ring)]
            + [pltpu.SemaphoreType.DMA for _ in range(2 * ring)]
        ),
    )
    def gather_kernel(idx_hbm, table_hbm, out_hbm, idx_v, *bufs_and_sems):
        bufs = list(bufs_and_sems[:ring])
        gsem = list(bufs_and_sems[ring:2 * ring])
        wsem = list(bufs_and_sems[2 * ring:])
        wid = lax.axis_index("s") * _NC + lax.axis_index("c")
        base = wid * n_per_w

        pltpu.sync_copy(idx_hbm.at[wid], idx_v)

        def issue_gather(chunk, slot):
            pltpu.async_copy(table_hbm.at[idx_v.at[chunk]], bufs[slot],
                             gsem[slot])

        def wait_gather(slot):
            pltpu.make_async_copy(table_hbm.at[idx_v.at[0]], bufs[slot],
                                  gsem[slot]).wait()

        def issue_write(chunk, slot):
            pltpu.async_copy(bufs[slot],
                             out_hbm.at[pl.ds(base + chunk * rows_per_chunk,
                                              rows_per_chunk)],
                             wsem[slot])

        def wait_write(slot):
            pltpu.make_async_copy(bufs[slot],
                                  out_hbm.at[pl.ds(base, rows_per_chunk)],
                                  wsem[slot]).wait()

        for b in range(ring - 1):
            issue_gather(b, b)

        def group(gr, _):
            for b in range(ring):
                chunk = gr * ring + b
                wait_gather(b)
                issue_write(chunk, b)
                refill = chunk + ring - 1
                slot = (b - 1) % ring
                if b == 0:
                    @pl.when(gr > 0)
                    def _():
                        wait_write(slot)
                    issue_gather(refill, slot)
                else:
                    @pl.when(refill < n_chunks)
                    def _():
                        wait_write(slot)
                        issue_gather(refill, slot)
            return ()

        lax.fori_loop(0, n_groups, group, (), unroll=False)

        for b in range(ring):
            wait_write(b)

    return gather_kernel


def kernel(indices, table):
    b, s = indices.shape
    v, d = table.shape
    n_total = b * s
    rows_per_chunk = 4
    ring = 4
    n_chunks = n_total // _NW // rows_per_chunk
    idx = jnp.asarray(indices, jnp.int32).reshape(_NW, n_chunks,
                                                  rows_per_chunk)
    gather = _build_gather(n_total, d, n_chunks, rows_per_chunk, ring)
    out = gather(idx, table)
    return out.reshape(b, s, d)
